# baseline (speedup 1.0000x reference)
import functools

import numpy as np
import jax
import jax.numpy as jnp
from jax import lax
from jax.experimental import pallas as pl
from jax.experimental.pallas import tpu as pltpu
from jax.experimental.pallas import tpu_sc as plsc

VOCAB = 1000000
EMBED = 64
SEQ = 200
BATCH = 4096

NC = 2
NS = 16
NW = NC * NS
TOTAL = BATCH * SEQ
ROWS_PER_W = TOTAL // NW
CHUNK = 800
NCHUNK = ROWS_PER_W // CHUNK
GATHER = 80
NGATHER = CHUNK // GATHER
LANES = 16


def _pe_table():
    pos = np.arange(SEQ, dtype=np.float32)[:, None]
    div = np.exp(np.arange(0, EMBED, 2, dtype=np.float32)
                 * (-(np.log(10000.0) / EMBED)))
    pe = np.zeros((SEQ, EMBED), dtype=np.float32)
    pe[:, 0::2] = np.sin(pos * div)
    pe[:, 1::2] = np.cos(pos * div)
    return pe


_PE = _pe_table()


def _sc_body(ids_hbm, pe_hbm, table_hbm, out_hbm, idx_v, rows_v, pe_v, gsem):
    wid = lax.axis_index("c") * NS + lax.axis_index("s")
    base = wid * ROWS_PER_W
    pltpu.sync_copy(pe_hbm, pe_v)

    @pl.loop(0, NCHUNK)
    def _chunk(ch):
        row0 = base + ch * CHUNK
        pltpu.sync_copy(ids_hbm.at[pl.ds(row0, CHUNK)], idx_v)
        copies = [
            pltpu.async_copy(
                table_hbm.at[idx_v.at[pl.ds(g * GATHER, GATHER)]],
                rows_v.at[pl.ds(g * GATHER, GATHER)],
                gsem,
            )
            for g in range(NGATHER)
        ]
        for c in copies:
            c.wait()

        @pl.loop(0, CHUNK // SEQ)
        def _seq(t):
            @pl.loop(0, SEQ)
            def _row(p):
                r = t * SEQ + p
                for q in range(EMBED // LANES):
                    sl = pl.ds(q * LANES, LANES)
                    rows_v[r, sl] = rows_v[r, sl] + pe_v[p, sl]

        pltpu.sync_copy(rows_v, out_hbm.at[pl.ds(row0, CHUNK)])


@jax.jit
def _encode(ids_flat, table, pe):
    mesh = plsc.VectorSubcoreMesh(
        core_axis_name="c", subcore_axis_name="s",
        num_cores=NC, num_subcores=NS,
    )
    out = pl.kernel(
        _sc_body,
        out_type=jax.ShapeDtypeStruct((TOTAL, EMBED), jnp.float32),
        mesh=mesh,
        scratch_types=[
            pltpu.VMEM((CHUNK,), jnp.int32),
            pltpu.VMEM((CHUNK, EMBED), jnp.float32),
            pltpu.VMEM((SEQ, EMBED), jnp.float32),
            pltpu.SemaphoreType.DMA,
        ],
        compiler_params=pltpu.CompilerParams(use_tc_tiling_on_sc=False),
    )(ids_flat, pe, table)
    return out.reshape(BATCH, SEQ, EMBED)


def kernel(input_ids, token_embedding):
    ids_flat = input_ids.reshape(-1).astype(jnp.int32)
    pe = jnp.asarray(_PE)
    return _encode(ids_flat, token_embedding, pe)
